# baseline (speedup 1.0000x reference)
import functools

import jax
import jax.numpy as jnp
from jax import lax
from jax.experimental import pallas as pl
from jax.experimental.pallas import tpu as pltpu
from jax.experimental.pallas import tpu_sc as plsc

N = 10000
E = 320000
NF = 4
D = 128
ND = 32
NL = 3
NG = 64
NC = 10
NW = 32
ROWS_PER_TILE = D // NW
EPS_ATT = E // 8
EC_ATT = 8000
ECL = 8000
NCH = E // ECL


def _prologue_body(xT_ref, lin0_WT_ref, aw1_ref, aw2_ref, lin0_b_ref,
                   att_b_ref, encWT_ref, encb_ref,
                   psrc_ref, pdst_ref, h0T_ref):
    xT = xT_ref[...]
    for f in range(NF):
        w0t = lin0_WT_ref[f]
        v1 = jnp.dot(aw1_ref[f], w0t, preferred_element_type=jnp.float32)
        v2 = jnp.dot(aw2_ref[f], w0t, preferred_element_type=jnp.float32)
        c1 = jnp.dot(aw1_ref[f], lin0_b_ref[f],
                     preferred_element_type=jnp.float32)
        c2 = jnp.dot(aw2_ref[f], lin0_b_ref[f],
                     preferred_element_type=jnp.float32)
        const = c1 + c2 + att_b_ref[f]
        psrc_ref[pl.ds(f, 1), :] = jnp.dot(
            v1, xT, preferred_element_type=jnp.float32)
        pdst_ref[pl.ds(f, 1), :] = jnp.dot(
            v2, xT, preferred_element_type=jnp.float32) + const
        h0T_ref[pl.ds(ND * f, ND), :] = jnp.dot(
            encWT_ref[f], xT, preferred_element_type=jnp.float32) \
            + encb_ref[f]


def _gru_factor(aggT_ref, hT_ref, wrelT_ref, brel_ref, wrootT_ref,
                wihT_ref, bih_ref, whhT_ref, bhh_ref, f):
    agg = aggT_ref[pl.ds(ND * f, ND), :]
    h = hT_ref[pl.ds(ND * f, ND), :]
    m = jnp.maximum(
        jnp.dot(wrelT_ref[f], agg, preferred_element_type=jnp.float32)
        + brel_ref[f]
        + jnp.dot(wrootT_ref[f], h, preferred_element_type=jnp.float32),
        0.0)
    gi = jnp.dot(wihT_ref[f], m,
                 preferred_element_type=jnp.float32) + bih_ref[f]
    gh = jnp.dot(whhT_ref[f], h,
                 preferred_element_type=jnp.float32) + bhh_ref[f]
    r = jax.nn.sigmoid(gi[0:ND] + gh[0:ND])
    z = jax.nn.sigmoid(gi[ND:2 * ND] + gh[ND:2 * ND])
    n = jnp.tanh(gi[2 * ND:3 * ND] + r * gh[2 * ND:3 * ND])
    return (1.0 - z) * n + z * h


def _layer_tc_body(aggT_ref, hT_ref, wrelT_ref, brel_ref, wrootT_ref,
                   wihT_ref, bih_ref, whhT_ref, bhh_ref, outT_ref):
    for f in range(NF):
        outT_ref[pl.ds(ND * f, ND), :] = _gru_factor(
            aggT_ref, hT_ref, wrelT_ref, brel_ref, wrootT_ref,
            wihT_ref, bih_ref, whhT_ref, bhh_ref, f)


def _final_tc_body(aggT_ref, hT_ref, wrelT_ref, brel_ref, wrootT_ref,
                   wihT_ref, bih_ref, whhT_ref, bhh_ref,
                   batch_ref, fc1WT_ref, fc1b_ref, fc2WT_ref, fc2b_ref,
                   meanT_ref, predT_ref):
    hT = jnp.concatenate(
        [_gru_factor(aggT_ref, hT_ref, wrelT_ref, brel_ref, wrootT_ref,
                     wihT_ref, bih_ref, whhT_ref, bhh_ref, f)
         for f in range(NF)], axis=0)
    io = lax.broadcasted_iota(jnp.int32, (N, NG), 1)
    oh = (io == batch_ref[...]).astype(jnp.float32)
    pooledT = jnp.dot(hT, oh, preferred_element_type=jnp.float32)
    cnt = jnp.sum(oh, axis=0, keepdims=True)
    meanT = pooledT / jnp.maximum(cnt, 1.0)
    meanT_ref[...] = meanT
    hid = jnp.maximum(
        jnp.dot(fc1WT_ref[...], meanT,
                preferred_element_type=jnp.float32) + fc1b_ref[...], 0.0)
    predT_ref[...] = jnp.dot(
        fc2WT_ref[...], hid,
        preferred_element_type=jnp.float32) + fc2b_ref[...]


def _att_body(psrc_hbm, pdst_hbm, src_hbm, dst_hbm,
              att_hbm, att16_hbm, sd_hbm,
              ps_v, pd_v, src_v, dst_v, ab_v, ab16_v, sd_v):
    wid = lax.axis_index("s") * 2 + lax.axis_index("c")
    f = wid // 8
    sl = wid % 8
    pltpu.sync_copy(psrc_hbm.at[pl.ds(f * N, N)], ps_v)
    pltpu.sync_copy(pdst_hbm.at[pl.ds(f * N, N)], pd_v)

    def chunk(ci, carry):
        base = sl * EPS_ATT + ci * EC_ATT
        pltpu.sync_copy(src_hbm.at[pl.ds(base, EC_ATT)], src_v)
        pltpu.sync_copy(dst_hbm.at[pl.ds(base, EC_ATT)], dst_v)

        @plsc.parallel_loop(0, EC_ATT // 32, unroll=5)
        def grp(p):
            ys = []
            for k in range(2):
                off = p * 32 + k * 16
                s16 = src_v[pl.ds(off, 16)]
                d16 = dst_v[pl.ds(off, 16)]
                a = plsc.load_gather(ps_v, [s16]) \
                    + plsc.load_gather(pd_v, [d16])
                y = 1.0 / (1.0 + jnp.exp(-6.0 * a))
                ab_v[pl.ds(off, 16)] = y
                ys.append(y)

                @pl.when(f == 0)
                def _():
                    sd_v[pl.ds(off, 16)] = s16 + lax.shift_left(d16, 14)

            ab16_v[pl.ds(p * 16, 16)] = plsc.bitcast(
                plsc.pack(ys[0], ys[1], format=plsc.PackFormat.INTERLEAVED),
                jnp.int32)

        pltpu.sync_copy(ab_v, att_hbm.at[pl.ds(f * E + base, EC_ATT)])
        half_base = f * (E // 2) + sl * (EPS_ATT // 2) + ci * (EC_ATT // 2)
        pltpu.sync_copy(ab16_v, att16_hbm.at[pl.ds(half_base, EC_ATT // 2)])

        @pl.when(f == 0)
        def _():
            pltpu.sync_copy(sd_v, sd_hbm.at[pl.ds(base, EC_ATT)])

        return carry

    lax.fori_loop(0, EPS_ATT // EC_ATT, chunk, 0)


def _layer_sc_body(hT_hbm, sd_hbm, att16_hbm, zeros_hbm, agg_hbm,
                   tab0, tab1, tab2, tab3, acc0, acc1, acc2, acc3,
                   sd0_v, ab0_v, sd1_v, ab1_v,
                   sem0, sem1):
    wid = lax.axis_index("s") * 2 + lax.axis_index("c")
    f = wid // 8
    base_row = wid * ROWS_PER_TILE * N
    tabs = (tab0, tab1, tab2, tab3)
    accs = (acc0, acc1, acc2, acc3)
    for r in range(ROWS_PER_TILE):
        pltpu.sync_copy(hT_hbm.at[pl.ds(base_row + r * N, N)], tabs[r])
        pltpu.sync_copy(zeros_hbm, accs[r])

    def start(ci, sdv, abv, sem):
        base = ci * ECL
        pltpu.async_copy(sd_hbm.at[pl.ds(base, ECL)], sdv, sem)
        pltpu.async_copy(
            att16_hbm.at[pl.ds(f * (E // 2) + ci * (ECL // 2), ECL // 2)],
            abv, sem)

    def wait(sdv, abv, sem):
        pltpu.make_async_copy(sd_hbm.at[pl.ds(0, ECL)], sdv, sem).wait()
        pltpu.make_async_copy(att16_hbm.at[pl.ds(0, ECL // 2)], abv,
                              sem).wait()

    def compute(sdv, abv):
        @plsc.parallel_loop(0, ECL // 32, unroll=10)
        def grp(p):
            ab = plsc.bitcast(abv[pl.ds(p * 16, 16)], jnp.bfloat16)
            y0, y1 = plsc.unpack(ab, format=plsc.PackFormat.INTERLEAVED)
            for k, yk in ((0, y0), (1, y1)):
                off = p * 32 + k * 16
                sd16 = sdv[pl.ds(off, 16)]
                s16 = jnp.bitwise_and(sd16, 16383)
                d16 = lax.shift_right_logical(sd16, 14)
                for r in range(ROWS_PER_TILE):
                    v = plsc.load_gather(tabs[r], [s16]) * yk
                    plsc.addupdate_scatter(accs[r], [d16], v)

    start(0, sd0_v, ab0_v, sem0)

    def outer(cc, carry):
        c0 = cc * 2
        start(c0 + 1, sd1_v, ab1_v, sem1)
        wait(sd0_v, ab0_v, sem0)
        compute(sd0_v, ab0_v)

        @pl.when(c0 + 2 < NCH)
        def _():
            start(c0 + 2, sd0_v, ab0_v, sem0)

        wait(sd1_v, ab1_v, sem1)
        compute(sd1_v, ab1_v)
        return carry

    lax.fori_loop(0, NCH // 2, outer, 0)
    for r in range(ROWS_PER_TILE):
        pltpu.sync_copy(accs[r], agg_hbm.at[pl.ds(base_row + r * N, N)])


@jax.jit
def kernel(x, edge_index, batch, params):
    f32 = jnp.float32
    src = edge_index[0].astype(jnp.int32)
    dst = edge_index[1].astype(jnp.int32)
    batch_i = batch.astype(jnp.int32).reshape(N, 1)

    xT = x.astype(f32).T
    lin0_WT = jnp.transpose(params["lin0_W"], (0, 2, 1))
    aw1 = params["att_W"][:, :D, 0][:, None, :]
    aw2 = params["att_W"][:, D:, 0][:, None, :]
    lin0_b = params["lin0_b"][:, :, None]
    att_b = params["att_b"][:, :, None]
    encWT = jnp.transpose(params["enc_lin_W"], (0, 2, 1))
    encb = params["enc_lin_b"][:, :, None]

    psrc, pdst, h0T = pl.pallas_call(
        _prologue_body,
        out_shape=(
            jax.ShapeDtypeStruct((NF, N), f32),
            jax.ShapeDtypeStruct((NF, N), f32),
            jax.ShapeDtypeStruct((D, N), f32),
        ),
    )(xT, lin0_WT, aw1, aw2, lin0_b, att_b, encWT, encb)

    mesh = plsc.VectorSubcoreMesh(core_axis_name="c", subcore_axis_name="s")

    sc_params = pltpu.CompilerParams(needs_layout_passes=False)

    att_flat, att16_flat, sd_packed = pl.kernel(
        _att_body,
        out_type=(
            jax.ShapeDtypeStruct((NF * E,), f32),
            jax.ShapeDtypeStruct((NF * E // 2,), jnp.int32),
            jax.ShapeDtypeStruct((E,), jnp.int32),
        ),
        mesh=mesh,
        compiler_params=sc_params,
        scratch_types=[
            pltpu.VMEM((N,), f32),
            pltpu.VMEM((N,), f32),
            pltpu.VMEM((EC_ATT,), jnp.int32),
            pltpu.VMEM((EC_ATT,), jnp.int32),
            pltpu.VMEM((EC_ATT,), f32),
            pltpu.VMEM((EC_ATT // 2,), jnp.int32),
            pltpu.VMEM((EC_ATT,), jnp.int32),
        ],
    )(psrc.reshape(NF * N), pdst.reshape(NF * N), src, dst)

    layer_sc = pl.kernel(
        _layer_sc_body,
        out_type=jax.ShapeDtypeStruct((D * N,), f32),
        mesh=mesh,
        compiler_params=sc_params,
        scratch_types=[
            pltpu.VMEM((N,), f32),
            pltpu.VMEM((N,), f32),
            pltpu.VMEM((N,), f32),
            pltpu.VMEM((N,), f32),
            pltpu.VMEM((N,), f32),
            pltpu.VMEM((N,), f32),
            pltpu.VMEM((N,), f32),
            pltpu.VMEM((N,), f32),
            pltpu.VMEM((ECL,), jnp.int32),
            pltpu.VMEM((ECL // 2,), jnp.int32),
            pltpu.VMEM((ECL,), jnp.int32),
            pltpu.VMEM((ECL // 2,), jnp.int32),
            pltpu.SemaphoreType.DMA,
            pltpu.SemaphoreType.DMA,
        ],
    )

    layer_tc = pl.pallas_call(
        _layer_tc_body,
        out_shape=jax.ShapeDtypeStruct((D, N), f32),
    )

    wihT = jnp.transpose(params["gru_Wih"], (0, 2, 1))
    bih = params["gru_bih"][:, :, None]
    whhT = jnp.transpose(params["gru_Whh"], (0, 2, 1))
    bhh = params["gru_bhh"][:, :, None]
    zeros_tile = jnp.zeros((N,), f32)

    def layer_weights(j):
        wrelT = jnp.transpose(params["conv_rel_W"][:, j], (0, 2, 1))
        brel = params["conv_rel_b"][:, j][:, :, None]
        wrootT = jnp.transpose(params["conv_root_W"][:, j], (0, 2, 1))
        return wrelT, brel, wrootT

    hT = h0T
    for j in range(NL - 1):
        aggT_flat = layer_sc(hT.reshape(D * N), sd_packed, att16_flat,
                             zeros_tile)
        wrelT, brel, wrootT = layer_weights(j)
        hT = layer_tc(aggT_flat.reshape(D, N), hT, wrelT, brel, wrootT,
                      wihT, bih, whhT, bhh)

    aggT_flat = layer_sc(hT.reshape(D * N), sd_packed, att16_flat,
                         zeros_tile)
    wrelT, brel, wrootT = layer_weights(NL - 1)
    meanT, predT = pl.pallas_call(
        _final_tc_body,
        out_shape=(
            jax.ShapeDtypeStruct((D, NG), f32),
            jax.ShapeDtypeStruct((NC, NG), f32),
        ),
    )(aggT_flat.reshape(D, N), hT, wrelT, brel, wrootT,
      wihT, bih, whhT, bhh, batch_i,
      params["fc1_W"].T, params["fc1_b"][:, None],
      params["fc2_W"].T, params["fc2_b"][:, None])

    pred = predT.T
    att = att_flat.reshape(NF, E)
    out_list = meanT.T.reshape(NG, NF, ND).transpose(1, 0, 2)
    return pred, att, out_list


# speedup vs baseline: 1.0317x; 1.0317x over previous
import functools

import jax
import jax.numpy as jnp
from jax import lax
from jax.experimental import pallas as pl
from jax.experimental.pallas import tpu as pltpu
from jax.experimental.pallas import tpu_sc as plsc

N = 10000
E = 320000
NF = 4
D = 128
ND = 32
NL = 3
NG = 64
NC = 10
NW = 32
ROWS_PER_TILE = D // NW
EPS_ATT = E // 8
EC_ATT = 8000
ECL = 8000
NCH = E // ECL


def _prologue_body(xT_ref, lin0_WT_ref, aw1_ref, aw2_ref, lin0_b_ref,
                   att_b_ref, encWT_ref, encb_ref,
                   psrc_ref, pdst_ref, h0T_ref):
    xT = xT_ref[...]
    for f in range(NF):
        w0t = lin0_WT_ref[f]
        v1 = jnp.dot(aw1_ref[f], w0t, preferred_element_type=jnp.float32)
        v2 = jnp.dot(aw2_ref[f], w0t, preferred_element_type=jnp.float32)
        c1 = jnp.dot(aw1_ref[f], lin0_b_ref[f],
                     preferred_element_type=jnp.float32)
        c2 = jnp.dot(aw2_ref[f], lin0_b_ref[f],
                     preferred_element_type=jnp.float32)
        const = c1 + c2 + att_b_ref[f]
        psrc_ref[pl.ds(f, 1), :] = jnp.dot(
            v1, xT, preferred_element_type=jnp.float32)
        pdst_ref[pl.ds(f, 1), :] = jnp.dot(
            v2, xT, preferred_element_type=jnp.float32) + const
        h0T_ref[pl.ds(ND * f, ND), :] = jnp.dot(
            encWT_ref[f], xT, preferred_element_type=jnp.float32) \
            + encb_ref[f]


def _gru_factor(aggT_ref, hT_ref, wrelT_ref, brel_ref, wrootT_ref,
                wihT_ref, bih_ref, whhT_ref, bhh_ref, f):
    agg = aggT_ref[pl.ds(ND * f, ND), :]
    h = hT_ref[pl.ds(ND * f, ND), :]
    m = jnp.maximum(
        jnp.dot(wrelT_ref[f], agg, preferred_element_type=jnp.float32)
        + brel_ref[f]
        + jnp.dot(wrootT_ref[f], h, preferred_element_type=jnp.float32),
        0.0)
    gi = jnp.dot(wihT_ref[f], m,
                 preferred_element_type=jnp.float32) + bih_ref[f]
    gh = jnp.dot(whhT_ref[f], h,
                 preferred_element_type=jnp.float32) + bhh_ref[f]
    r = jax.nn.sigmoid(gi[0:ND] + gh[0:ND])
    z = jax.nn.sigmoid(gi[ND:2 * ND] + gh[ND:2 * ND])
    n = jnp.tanh(gi[2 * ND:3 * ND] + r * gh[2 * ND:3 * ND])
    return (1.0 - z) * n + z * h


def _layer_tc_body(aggT_ref, hT_ref, wrelT_ref, brel_ref, wrootT_ref,
                   wihT_ref, bih_ref, whhT_ref, bhh_ref, outT_ref):
    for f in range(NF):
        outT_ref[pl.ds(ND * f, ND), :] = _gru_factor(
            aggT_ref, hT_ref, wrelT_ref, brel_ref, wrootT_ref,
            wihT_ref, bih_ref, whhT_ref, bhh_ref, f)


def _final_tc_body(aggT_ref, hT_ref, wrelT_ref, brel_ref, wrootT_ref,
                   wihT_ref, bih_ref, whhT_ref, bhh_ref,
                   batch_ref, fc1WT_ref, fc1b_ref, fc2WT_ref, fc2b_ref,
                   meanT_ref, predT_ref):
    hT = jnp.concatenate(
        [_gru_factor(aggT_ref, hT_ref, wrelT_ref, brel_ref, wrootT_ref,
                     wihT_ref, bih_ref, whhT_ref, bhh_ref, f)
         for f in range(NF)], axis=0)
    io = lax.broadcasted_iota(jnp.int32, (N, NG), 1)
    oh = (io == batch_ref[...]).astype(jnp.float32)
    pooledT = jnp.dot(hT, oh, preferred_element_type=jnp.float32)
    cnt = jnp.sum(oh, axis=0, keepdims=True)
    meanT = pooledT / jnp.maximum(cnt, 1.0)
    meanT_ref[...] = meanT
    hid = jnp.maximum(
        jnp.dot(fc1WT_ref[...], meanT,
                preferred_element_type=jnp.float32) + fc1b_ref[...], 0.0)
    predT_ref[...] = jnp.dot(
        fc2WT_ref[...], hid,
        preferred_element_type=jnp.float32) + fc2b_ref[...]


def _att_body(psrc_hbm, pdst_hbm, src_hbm, dst_hbm,
              att_hbm, att16_hbm, sd_hbm,
              ps_v, pd_v, s0_v, d0_v, s1_v, d1_v,
              ab0_v, ab16_0v, sd0_v, ab1_v, ab16_1v, sd1_v,
              sem_i0, sem_i1, sem_o0, sem_o1):
    wid = lax.axis_index("s") * 2 + lax.axis_index("c")
    f = wid // 8
    sl = wid % 8
    pltpu.sync_copy(psrc_hbm.at[pl.ds(f * N, N)], ps_v)
    pltpu.sync_copy(pdst_hbm.at[pl.ds(f * N, N)], pd_v)

    NCA = EPS_ATT // EC_ATT
    srcs = ((s0_v, d0_v, sem_i0), (s1_v, d1_v, sem_i1))
    dsts = ((ab0_v, ab16_0v, sd0_v, sem_o0), (ab1_v, ab16_1v, sd1_v, sem_o1))

    def start_in(c, b):
        base = sl * EPS_ATT + c * EC_ATT
        sv, dv, sem = srcs[b]
        pltpu.async_copy(src_hbm.at[pl.ds(base, EC_ATT)], sv, sem)
        pltpu.async_copy(dst_hbm.at[pl.ds(base, EC_ATT)], dv, sem)

    def wait_in(b):
        sv, dv, sem = srcs[b]
        pltpu.make_async_copy(src_hbm.at[pl.ds(0, EC_ATT)], sv, sem).wait()
        pltpu.make_async_copy(dst_hbm.at[pl.ds(0, EC_ATT)], dv, sem).wait()

    def start_out(c, b):
        base = sl * EPS_ATT + c * EC_ATT
        abv, ab16v, sdv, sem = dsts[b]
        pltpu.async_copy(abv, att_hbm.at[pl.ds(f * E + base, EC_ATT)], sem)
        half_base = f * (E // 2) + sl * (EPS_ATT // 2) + c * (EC_ATT // 2)
        pltpu.async_copy(
            ab16v, att16_hbm.at[pl.ds(half_base, EC_ATT // 2)], sem)

        @pl.when(f == 0)
        def _():
            pltpu.async_copy(sdv, sd_hbm.at[pl.ds(base, EC_ATT)], sem)

    def wait_out(b):
        abv, ab16v, sdv, sem = dsts[b]
        pltpu.make_async_copy(abv, att_hbm.at[pl.ds(0, EC_ATT)], sem).wait()
        pltpu.make_async_copy(
            ab16v, att16_hbm.at[pl.ds(0, EC_ATT // 2)], sem).wait()

        @pl.when(f == 0)
        def _():
            pltpu.make_async_copy(sdv, sd_hbm.at[pl.ds(0, EC_ATT)],
                                  sem).wait()

    def compute(b):
        sv, dv, _ = srcs[b]
        abv, ab16v, sdv, _ = dsts[b]

        @plsc.parallel_loop(0, EC_ATT // 32, unroll=5)
        def grp(p):
            ys = []
            for k in range(2):
                off = p * 32 + k * 16
                s16 = sv[pl.ds(off, 16)]
                d16 = dv[pl.ds(off, 16)]
                a = plsc.load_gather(ps_v, [s16]) \
                    + plsc.load_gather(pd_v, [d16])
                y = 1.0 / (1.0 + jnp.exp(-6.0 * a))
                abv[pl.ds(off, 16)] = y
                ys.append(y)

                @pl.when(f == 0)
                def _():
                    sdv[pl.ds(off, 16)] = s16 + lax.shift_left(d16, 14)

            ab16v[pl.ds(p * 16, 16)] = plsc.bitcast(
                plsc.pack(ys[0], ys[1], format=plsc.PackFormat.INTERLEAVED),
                jnp.int32)

    start_in(0, 0)
    for c in range(NCA):
        b = c % 2
        if c + 1 < NCA:
            start_in(c + 1, 1 - b)
        wait_in(b)
        if c >= 2:
            wait_out(b)
        compute(b)
        start_out(c, b)
    wait_out((NCA - 2) % 2)
    wait_out((NCA - 1) % 2)


def _layer_sc_body(hT_hbm, sd_hbm, att16_hbm, zeros_hbm, agg_hbm,
                   tab0, tab1, tab2, tab3, acc0, acc1, acc2, acc3,
                   sd0_v, ab0_v, sd1_v, ab1_v,
                   sem0, sem1):
    wid = lax.axis_index("s") * 2 + lax.axis_index("c")
    f = wid // 8
    base_row = wid * ROWS_PER_TILE * N
    tabs = (tab0, tab1, tab2, tab3)
    accs = (acc0, acc1, acc2, acc3)
    for r in range(ROWS_PER_TILE):
        pltpu.sync_copy(hT_hbm.at[pl.ds(base_row + r * N, N)], tabs[r])
        pltpu.sync_copy(zeros_hbm, accs[r])

    def start(ci, sdv, abv, sem):
        base = ci * ECL
        pltpu.async_copy(sd_hbm.at[pl.ds(base, ECL)], sdv, sem)
        pltpu.async_copy(
            att16_hbm.at[pl.ds(f * (E // 2) + ci * (ECL // 2), ECL // 2)],
            abv, sem)

    def wait(sdv, abv, sem):
        pltpu.make_async_copy(sd_hbm.at[pl.ds(0, ECL)], sdv, sem).wait()
        pltpu.make_async_copy(att16_hbm.at[pl.ds(0, ECL // 2)], abv,
                              sem).wait()

    def compute(sdv, abv):
        @plsc.parallel_loop(0, ECL // 32, unroll=5)
        def grp(p):
            ab = plsc.bitcast(abv[pl.ds(p * 16, 16)], jnp.bfloat16)
            y0, y1 = plsc.unpack(ab, format=plsc.PackFormat.INTERLEAVED)
            for k, yk in ((0, y0), (1, y1)):
                off = p * 32 + k * 16
                sd16 = sdv[pl.ds(off, 16)]
                s16 = jnp.bitwise_and(sd16, 16383)
                d16 = lax.shift_right_logical(sd16, 14)
                for r in range(ROWS_PER_TILE):
                    v = plsc.load_gather(tabs[r], [s16]) * yk
                    plsc.addupdate_scatter(accs[r], [d16], v)

    start(0, sd0_v, ab0_v, sem0)

    def outer(cc, carry):
        c0 = cc * 2
        start(c0 + 1, sd1_v, ab1_v, sem1)
        wait(sd0_v, ab0_v, sem0)
        compute(sd0_v, ab0_v)

        @pl.when(c0 + 2 < NCH)
        def _():
            start(c0 + 2, sd0_v, ab0_v, sem0)

        wait(sd1_v, ab1_v, sem1)
        compute(sd1_v, ab1_v)
        return carry

    lax.fori_loop(0, NCH // 2, outer, 0)
    for r in range(ROWS_PER_TILE):
        pltpu.sync_copy(accs[r], agg_hbm.at[pl.ds(base_row + r * N, N)])


@jax.jit
def kernel(x, edge_index, batch, params):
    f32 = jnp.float32
    src = edge_index[0].astype(jnp.int32)
    dst = edge_index[1].astype(jnp.int32)
    batch_i = batch.astype(jnp.int32).reshape(N, 1)

    xT = x.astype(f32).T
    lin0_WT = jnp.transpose(params["lin0_W"], (0, 2, 1))
    aw1 = params["att_W"][:, :D, 0][:, None, :]
    aw2 = params["att_W"][:, D:, 0][:, None, :]
    lin0_b = params["lin0_b"][:, :, None]
    att_b = params["att_b"][:, :, None]
    encWT = jnp.transpose(params["enc_lin_W"], (0, 2, 1))
    encb = params["enc_lin_b"][:, :, None]

    psrc, pdst, h0T = pl.pallas_call(
        _prologue_body,
        out_shape=(
            jax.ShapeDtypeStruct((NF, N), f32),
            jax.ShapeDtypeStruct((NF, N), f32),
            jax.ShapeDtypeStruct((D, N), f32),
        ),
    )(xT, lin0_WT, aw1, aw2, lin0_b, att_b, encWT, encb)

    mesh = plsc.VectorSubcoreMesh(core_axis_name="c", subcore_axis_name="s")

    sc_params = pltpu.CompilerParams(needs_layout_passes=False)

    att_flat, att16_flat, sd_packed = pl.kernel(
        _att_body,
        out_type=(
            jax.ShapeDtypeStruct((NF * E,), f32),
            jax.ShapeDtypeStruct((NF * E // 2,), jnp.int32),
            jax.ShapeDtypeStruct((E,), jnp.int32),
        ),
        mesh=mesh,
        compiler_params=sc_params,
        scratch_types=[
            pltpu.VMEM((N,), f32),
            pltpu.VMEM((N,), f32),
            pltpu.VMEM((EC_ATT,), jnp.int32),
            pltpu.VMEM((EC_ATT,), jnp.int32),
            pltpu.VMEM((EC_ATT,), jnp.int32),
            pltpu.VMEM((EC_ATT,), jnp.int32),
            pltpu.VMEM((EC_ATT,), f32),
            pltpu.VMEM((EC_ATT // 2,), jnp.int32),
            pltpu.VMEM((EC_ATT,), jnp.int32),
            pltpu.VMEM((EC_ATT,), f32),
            pltpu.VMEM((EC_ATT // 2,), jnp.int32),
            pltpu.VMEM((EC_ATT,), jnp.int32),
            pltpu.SemaphoreType.DMA,
            pltpu.SemaphoreType.DMA,
            pltpu.SemaphoreType.DMA,
            pltpu.SemaphoreType.DMA,
        ],
    )(psrc.reshape(NF * N), pdst.reshape(NF * N), src, dst)

    layer_sc = pl.kernel(
        _layer_sc_body,
        out_type=jax.ShapeDtypeStruct((D * N,), f32),
        mesh=mesh,
        compiler_params=sc_params,
        scratch_types=[
            pltpu.VMEM((N,), f32),
            pltpu.VMEM((N,), f32),
            pltpu.VMEM((N,), f32),
            pltpu.VMEM((N,), f32),
            pltpu.VMEM((N,), f32),
            pltpu.VMEM((N,), f32),
            pltpu.VMEM((N,), f32),
            pltpu.VMEM((N,), f32),
            pltpu.VMEM((ECL,), jnp.int32),
            pltpu.VMEM((ECL // 2,), jnp.int32),
            pltpu.VMEM((ECL,), jnp.int32),
            pltpu.VMEM((ECL // 2,), jnp.int32),
            pltpu.SemaphoreType.DMA,
            pltpu.SemaphoreType.DMA,
        ],
    )

    layer_tc = pl.pallas_call(
        _layer_tc_body,
        out_shape=jax.ShapeDtypeStruct((D, N), f32),
    )

    wihT = jnp.transpose(params["gru_Wih"], (0, 2, 1))
    bih = params["gru_bih"][:, :, None]
    whhT = jnp.transpose(params["gru_Whh"], (0, 2, 1))
    bhh = params["gru_bhh"][:, :, None]
    zeros_tile = jnp.zeros((N,), f32)

    def layer_weights(j):
        wrelT = jnp.transpose(params["conv_rel_W"][:, j], (0, 2, 1))
        brel = params["conv_rel_b"][:, j][:, :, None]
        wrootT = jnp.transpose(params["conv_root_W"][:, j], (0, 2, 1))
        return wrelT, brel, wrootT

    hT = h0T
    for j in range(NL - 1):
        aggT_flat = layer_sc(hT.reshape(D * N), sd_packed, att16_flat,
                             zeros_tile)
        wrelT, brel, wrootT = layer_weights(j)
        hT = layer_tc(aggT_flat.reshape(D, N), hT, wrelT, brel, wrootT,
                      wihT, bih, whhT, bhh)

    aggT_flat = layer_sc(hT.reshape(D * N), sd_packed, att16_flat,
                         zeros_tile)
    wrelT, brel, wrootT = layer_weights(NL - 1)
    meanT, predT = pl.pallas_call(
        _final_tc_body,
        out_shape=(
            jax.ShapeDtypeStruct((D, NG), f32),
            jax.ShapeDtypeStruct((NC, NG), f32),
        ),
    )(aggT_flat.reshape(D, N), hT, wrelT, brel, wrootT,
      wihT, bih, whhT, bhh, batch_i,
      params["fc1_W"].T, params["fc1_b"][:, None],
      params["fc2_W"].T, params["fc2_b"][:, None])

    pred = predT.T
    att = att_flat.reshape(NF, E)
    out_list = meanT.T.reshape(NG, NF, ND).transpose(1, 0, 2)
    return pred, att, out_list
